# baseline (speedup 1.0000x reference)
import jax
import jax.numpy as jnp
from jax import lax
from jax.experimental import pallas as pl
from jax.experimental.pallas import tpu as pltpu
from jax.experimental.pallas import tpu_sc as plsc

B = 16384
L = 50
D_SPARSE = 3
D_DENSE = 2
D_OUT = 2

NC = 2
NS = 16
NW = NC * NS
BPW = B // NW
PASSES = 4
BPP = BPW // PASSES
IPP = BPP * L
WPP = IPP * D_SPARSE


def _sc_kernel(idx3_hbm, seg3_hbm, dense_hbm, wsplat_hbm, zeros_hbm, flat_hbm,
               out_hbm, idx3_v, seg3_v, words_v, acc_v, dense_v, wsplat_v,
               out_v, shared_acc, sem):
    c = lax.axis_index("c")
    s = lax.axis_index("s")
    w = s * NC + c

    pltpu.sync_copy(dense_hbm.at[w], dense_v)
    pltpu.sync_copy(wsplat_hbm, wsplat_v)
    pltpu.sync_copy(zeros_hbm, shared_acc.at[pl.ds(s * BPW * D_SPARSE,
                                                   BPW * D_SPARSE)])

    for p in range(PASSES):
        pltpu.sync_copy(idx3_hbm.at[w, p], idx3_v)
        pltpu.sync_copy(seg3_hbm.at[s, p], seg3_v)
        pltpu.async_copy(flat_hbm.at[idx3_v], words_v, sem).wait()
        pltpu.sync_copy(words_v, shared_acc.at[seg3_v], add=True)

    pltpu.sync_copy(shared_acc.at[pl.ds(s * BPW * D_SPARSE, BPW * D_SPARSE)],
                    acc_v)

    lanes = lax.iota(jnp.int32, 16)
    wv = [wsplat_v[i, :] for i in range(2 * (D_DENSE + D_SPARSE) + D_OUT)]
    for g in range(BPW // 16):
        r = lanes + g * 16
        d0 = plsc.load_gather(dense_v, [r * D_DENSE])
        d1 = plsc.load_gather(dense_v, [r * D_DENSE + 1])
        m0 = plsc.load_gather(acc_v, [r * D_SPARSE])
        m1 = plsc.load_gather(acc_v, [r * D_SPARSE + 1])
        m2 = plsc.load_gather(acc_v, [r * D_SPARSE + 2])
        o0 = d0 * wv[0] + d1 * wv[1] + m0 * wv[2] + m1 * wv[3] + m2 * wv[4] + wv[10]
        o1 = d0 * wv[5] + d1 * wv[6] + m0 * wv[7] + m1 * wv[8] + m2 * wv[9] + wv[11]
        plsc.store_scatter(out_v, [r * D_OUT], o0)
        plsc.store_scatter(out_v, [r * D_OUT + 1], o1)

    pltpu.sync_copy(out_v, out_hbm.at[w])


@jax.jit
def kernel(dense_features, sparse_features, em_weight, W, b):
    idx = sparse_features.astype(jnp.int32)
    idx3 = (idx.reshape(-1)[:, None] * D_SPARSE
            + jnp.arange(D_SPARSE, dtype=jnp.int32)[None, :]
            ).reshape(NW, PASSES, WPP)
    bloc = jnp.arange(WPP, dtype=jnp.int32) // (L * D_SPARSE)
    d = jnp.arange(WPP, dtype=jnp.int32) % D_SPARSE
    seg3 = ((jnp.arange(NS, dtype=jnp.int32)[:, None, None] * BPW
             + jnp.arange(PASSES, dtype=jnp.int32)[None, :, None] * BPP
             + bloc[None, None, :]) * D_SPARSE
            + d[None, None, :])
    dense_flat = dense_features.reshape(NW, BPW * D_DENSE)
    scale = jnp.array([1.0, 1.0, 1.0 / L, 1.0 / L, 1.0 / L], jnp.float32)
    wf = (W * scale[None, :]).reshape(-1)
    wsplat = jnp.tile(jnp.concatenate([wf, b])[:, None], (1, 16))
    zeros = jnp.zeros((BPW * D_SPARSE,), jnp.float32)
    flat = em_weight.reshape(-1)

    mesh = plsc.VectorSubcoreMesh(core_axis_name="c", subcore_axis_name="s")
    run = pl.kernel(
        _sc_kernel,
        mesh=mesh,
        compiler_params=pltpu.CompilerParams(
            needs_layout_passes=False, use_tc_tiling_on_sc=False),
        out_type=jax.ShapeDtypeStruct((NW, BPW * D_OUT), jnp.float32),
        scratch_types=[
            pltpu.VMEM((WPP,), jnp.int32),
            pltpu.VMEM((WPP,), jnp.int32),
            pltpu.VMEM((WPP,), jnp.float32),
            pltpu.VMEM((BPW * D_SPARSE,), jnp.float32),
            pltpu.VMEM((BPW * D_DENSE,), jnp.float32),
            pltpu.VMEM((12, 16), jnp.float32),
            pltpu.VMEM((BPW * D_OUT,), jnp.float32),
            pltpu.VMEM_SHARED((NS * BPW * D_SPARSE,), jnp.float32),
            pltpu.SemaphoreType.DMA,
        ],
    )
    out = run(idx3, seg3, dense_flat, wsplat, zeros, flat)
    return out.reshape(B, D_OUT)


# speedup vs baseline: 1.0080x; 1.0080x over previous
import jax
import jax.numpy as jnp
from jax import lax
from jax.experimental import pallas as pl
from jax.experimental.pallas import tpu as pltpu
from jax.experimental.pallas import tpu_sc as plsc

B = 16384
L = 50
D_SPARSE = 3
D_DENSE = 2
D_OUT = 2

NC = 2
NS = 16
NW = NC * NS
BPW = B // NW
PASSES = 4
BPP = BPW // PASSES
WPL = BPP * D_SPARSE
WPP = WPL * L


def _sc_kernel(idx3_hbm, dense_hbm, wsplat_hbm, flat_hbm,
               out_hbm, idx3_v, words_v, acc_v, dense_v, wsplat_v,
               out_v, sem):
    c = lax.axis_index("c")
    s = lax.axis_index("s")
    w = s * NC + c

    pltpu.sync_copy(dense_hbm.at[w], dense_v)
    pltpu.sync_copy(wsplat_hbm, wsplat_v)

    for p in range(PASSES):
        pltpu.sync_copy(idx3_hbm.at[w, p], idx3_v)
        pltpu.async_copy(flat_hbm.at[idx3_v], words_v, sem).wait()
        for k in range(WPL // 16):
            def body(l, acc):
                return acc + words_v[pl.ds(l * WPL + k * 16, 16)]
            acc = lax.fori_loop(1, L, body, words_v[pl.ds(k * 16, 16)])
            acc_v[pl.ds(p * WPL + k * 16, 16)] = acc

    lanes = lax.iota(jnp.int32, 16)
    wv = [wsplat_v[i, :] for i in range(2 * (D_DENSE + D_SPARSE) + D_OUT)]
    for g in range(BPW // 16):
        r = lanes + g * 16
        d0 = plsc.load_gather(dense_v, [r * D_DENSE])
        d1 = plsc.load_gather(dense_v, [r * D_DENSE + 1])
        m0 = plsc.load_gather(acc_v, [r * D_SPARSE])
        m1 = plsc.load_gather(acc_v, [r * D_SPARSE + 1])
        m2 = plsc.load_gather(acc_v, [r * D_SPARSE + 2])
        o0 = d0 * wv[0] + d1 * wv[1] + m0 * wv[2] + m1 * wv[3] + m2 * wv[4] + wv[10]
        o1 = d0 * wv[5] + d1 * wv[6] + m0 * wv[7] + m1 * wv[8] + m2 * wv[9] + wv[11]
        plsc.store_scatter(out_v, [r * D_OUT], o0)
        plsc.store_scatter(out_v, [r * D_OUT + 1], o1)

    pltpu.sync_copy(out_v, out_hbm.at[w])


@jax.jit
def kernel(dense_features, sparse_features, em_weight, W, b):
    idx = sparse_features.astype(jnp.int32)
    idx_lm = idx.reshape(NW, PASSES, BPP, L).transpose(0, 1, 3, 2)
    idx3 = (idx_lm[..., None] * D_SPARSE
            + jnp.arange(D_SPARSE, dtype=jnp.int32)
            ).reshape(NW, PASSES, WPP)
    dense_flat = dense_features.reshape(NW, BPW * D_DENSE)
    scale = jnp.array([1.0, 1.0, 1.0 / L, 1.0 / L, 1.0 / L], jnp.float32)
    wf = (W * scale[None, :]).reshape(-1)
    wsplat = jnp.tile(jnp.concatenate([wf, b])[:, None], (1, 16))
    flat = em_weight.reshape(-1)

    mesh = plsc.VectorSubcoreMesh(core_axis_name="c", subcore_axis_name="s")
    run = pl.kernel(
        _sc_kernel,
        mesh=mesh,
        compiler_params=pltpu.CompilerParams(
            needs_layout_passes=False, use_tc_tiling_on_sc=False),
        out_type=jax.ShapeDtypeStruct((NW, BPW * D_OUT), jnp.float32),
        scratch_types=[
            pltpu.VMEM((WPP,), jnp.int32),
            pltpu.VMEM((WPP,), jnp.float32),
            pltpu.VMEM((BPW * D_SPARSE,), jnp.float32),
            pltpu.VMEM((BPW * D_DENSE,), jnp.float32),
            pltpu.VMEM((12, 16), jnp.float32),
            pltpu.VMEM((BPW * D_OUT,), jnp.float32),
            pltpu.SemaphoreType.DMA,
        ],
    )
    out = run(idx3, dense_flat, wsplat, flat)
    return out.reshape(B, D_OUT)


# speedup vs baseline: 14.3920x; 14.2785x over previous
import jax
import jax.numpy as jnp
from jax import lax
from jax.experimental import pallas as pl
from jax.experimental.pallas import tpu as pltpu
from jax.experimental.pallas import tpu_sc as plsc

B = 16384
L = 50
V = 1000000
D_SPARSE = 3
D_DENSE = 2
D_OUT = 2

NC = 2
NS = 16
NW = NC * NS
BPW = B // NW
PASSES = 4
BPP = BPW // PASSES
IPP = BPP * L
WPP = IPP * D_SPARSE
VPW = 31248
VTAIL = V - VPW * NW

_MESH = plsc.VectorSubcoreMesh(core_axis_name="c", subcore_axis_name="s")
_CP = pltpu.CompilerParams(needs_layout_passes=False, use_tc_tiling_on_sc=False)


def _relayout_kernel(emt_hbm, flat_hbm, col_v, tail_v):
    c = lax.axis_index("c")
    s = lax.axis_index("s")
    w = s * NC + c
    for d in range(D_SPARSE):
        pltpu.sync_copy(emt_hbm.at[d, pl.ds(w * VPW, VPW)], col_v)
        pltpu.sync_copy(col_v, flat_hbm.at[pl.ds(d * V + w * VPW, VPW)])

    @pl.when(w == NW - 1)
    def _():
        for d in range(D_SPARSE):
            pltpu.sync_copy(emt_hbm.at[d, pl.ds(NW * VPW, VTAIL)], tail_v)
            pltpu.sync_copy(tail_v, flat_hbm.at[pl.ds(d * V + NW * VPW, VTAIL)])


def _main_kernel(sp_hbm, dn_hbm, wsplat_hbm, flat_hbm, out_hbm,
                 sp_v, dn_v, wsplat_v, idx3_v, words_v, acc_v, out_v, sem):
    c = lax.axis_index("c")
    s = lax.axis_index("s")
    w = s * NC + c
    lanes = lax.iota(jnp.int32, 16)

    pltpu.sync_copy(sp_hbm.at[pl.ds(w * BPW, BPW), :], sp_v)
    pltpu.sync_copy(dn_hbm.at[pl.ds(w * BPW, BPW), :], dn_v)
    pltpu.sync_copy(wsplat_hbm, wsplat_v)

    for p in range(PASSES):
        def build(k, _):
            q16 = k * 16 + lanes
            q = q16 + p * IPP
            vidx = plsc.load_gather(sp_v, [q // L, q % L])
            j3 = q16 * D_SPARSE
            plsc.store_scatter(idx3_v, [j3], vidx)
            plsc.store_scatter(idx3_v, [j3 + 1], vidx + V)
            plsc.store_scatter(idx3_v, [j3 + 2], vidx + 2 * V)
            return 0
        lax.fori_loop(0, IPP // 16, build, 0)

        pltpu.async_copy(flat_hbm.at[idx3_v], words_v, sem).wait()

        for g in range(BPP // 16):
            base = (g * 16 + lanes) * (L * D_SPARSE)
            for d in range(D_SPARSE):
                def rbody(l, acc):
                    return acc + plsc.load_gather(
                        words_v, [base + (l * D_SPARSE + d)])
                acc = lax.fori_loop(
                    1, L, rbody, plsc.load_gather(words_v, [base + d]))
                plsc.store_scatter(
                    acc_v, [(p * BPP + g * 16 + lanes) * D_SPARSE + d], acc)

    wv = [wsplat_v[i, :] for i in range(2 * (D_DENSE + D_SPARSE) + D_OUT)]
    zero16 = jnp.zeros((16,), jnp.int32)
    for g in range(BPW // 16):
        r = lanes + g * 16
        d0 = plsc.load_gather(dn_v, [r, zero16])
        d1 = plsc.load_gather(dn_v, [r, zero16 + 1])
        m0 = plsc.load_gather(acc_v, [r * D_SPARSE])
        m1 = plsc.load_gather(acc_v, [r * D_SPARSE + 1])
        m2 = plsc.load_gather(acc_v, [r * D_SPARSE + 2])
        o0 = d0 * wv[0] + d1 * wv[1] + m0 * wv[2] + m1 * wv[3] + m2 * wv[4] + wv[10]
        o1 = d0 * wv[5] + d1 * wv[6] + m0 * wv[7] + m1 * wv[8] + m2 * wv[9] + wv[11]
        plsc.store_scatter(out_v, [r * D_OUT], o0)
        plsc.store_scatter(out_v, [r * D_OUT + 1], o1)

    pltpu.sync_copy(out_v, out_hbm.at[w])


@jax.jit
def kernel(dense_features, sparse_features, em_weight, W, b):
    sp = sparse_features.astype(jnp.int32)
    scale = jnp.array([1.0, 1.0, 1.0 / L, 1.0 / L, 1.0 / L], jnp.float32)
    wf = (W * scale[None, :]).reshape(-1)
    wsplat = jnp.tile(jnp.concatenate([wf, b])[:, None], (1, 16))

    relayout = pl.kernel(
        _relayout_kernel,
        mesh=_MESH,
        compiler_params=_CP,
        out_type=jax.ShapeDtypeStruct((D_SPARSE * V,), jnp.float32),
        scratch_types=[pltpu.VMEM((VPW,), jnp.float32),
                       pltpu.VMEM((VTAIL,), jnp.float32)],
    )
    flat = relayout(em_weight.T)

    run = pl.kernel(
        _main_kernel,
        mesh=_MESH,
        compiler_params=_CP,
        out_type=jax.ShapeDtypeStruct((NW, BPW * D_OUT), jnp.float32),
        scratch_types=[
            pltpu.VMEM((BPW, L), jnp.int32),
            pltpu.VMEM((BPW, D_DENSE), jnp.float32),
            pltpu.VMEM((12, 16), jnp.float32),
            pltpu.VMEM((WPP,), jnp.int32),
            pltpu.VMEM((WPP,), jnp.float32),
            pltpu.VMEM((BPW * D_SPARSE,), jnp.float32),
            pltpu.VMEM((BPW * D_OUT,), jnp.float32),
            pltpu.SemaphoreType.DMA,
        ],
    )
    out = run(sp, dense_features, wsplat, flat)
    return out.reshape(B, D_OUT)
